# baseline (speedup 1.0000x reference)
import functools

import jax
import jax.numpy as jnp
from jax import lax
from jax.experimental import pallas as pl
from jax.experimental.pallas import tpu as pltpu
from jax.experimental.pallas import tpu_sc as plsc

N = 100000
E = 3200000
D = 128
C = 16
HID = 64
K = 10
ALPHA = 0.1

NC = 2
NS = 16
NW = NC * NS

NP = 100096
ROWS_PER_SUB = NP // NS
GPW = 784
EP = NW * GPW * 128
NG = EP // 128
GSTEP = 4
NQUAD = GPW // (4 * GSTEP)
NGA = NG + 8
NPAIR = GPW // (2 * GSTEP)
DCH = 92
NDP = (ROWS_PER_SUB // DCH) // 2
LANE_ROWS = NP * C // 128


def _zero_acc(zeros_hbm, acc, row0):
    pltpu.sync_copy(zeros_hbm.at[pl.ds(0, ROWS_PER_SUB)],
                    acc.at[pl.ds(row0, ROWS_PER_SUB)])


def _idx4(src_hbm, dst_hbm, b0, b1, sbuf0, dbuf0, sbuf1, dbuf1, sem):
    return [
        pltpu.make_async_copy(src_hbm.at[pl.ds(b0, GSTEP)], sbuf0, sem),
        pltpu.make_async_copy(dst_hbm.at[pl.ds(b0, GSTEP)], dbuf0, sem),
        pltpu.make_async_copy(src_hbm.at[pl.ds(b1, GSTEP)], sbuf1, sem),
        pltpu.make_async_copy(dst_hbm.at[pl.ds(b1, GSTEP)], dbuf1, sem),
    ]


def _edge_phase(u_hbm, src_hbm, dst_hbm, acc,
                siA0, diA0, siA1, diA1, siB0, diB0, siB1, diB1,
                rb0, rb1, sem_g0, sem_g1, sem_s, sem_ia, sem_ib, g0):
    setA = (siA0, diA0, siA1, diA1)
    setB = (siB0, diB0, siB1, diB1)

    for d in _idx4(src_hbm, dst_hbm, g0, g0 + GSTEP, *setA, sem_ia):
        d.start()

    def quad(t, carry):
        bA0 = g0 + t * 4 * GSTEP
        bB0 = bA0 + 2 * GSTEP
        for d in _idx4(src_hbm, dst_hbm, bA0, bA0 + GSTEP, *setA, sem_ia):
            d.wait()
        for d in _idx4(src_hbm, dst_hbm, bB0, bB0 + GSTEP, *setB, sem_ib):
            d.start()
        gA0 = [pltpu.async_copy(u_hbm.at[siA0.at[j]],
                                rb0.at[pl.ds(j * 128, 128)], sem_g0)
               for j in range(GSTEP)]
        gA1 = [pltpu.async_copy(u_hbm.at[siA1.at[j]],
                                rb1.at[pl.ds(j * 128, 128)], sem_g1)
               for j in range(GSTEP)]
        for d in gA0:
            d.wait()
        sA0 = [pltpu.async_copy(rb0.at[pl.ds(j * 128, 128)],
                                acc.at[diA0.at[j]], sem_s, add=True)
               for j in range(GSTEP)]
        for d in gA1:
            d.wait()
        sA1 = [pltpu.async_copy(rb1.at[pl.ds(j * 128, 128)],
                                acc.at[diA1.at[j]], sem_s, add=True)
               for j in range(GSTEP)]
        for d in sA0 + sA1:
            d.wait()
        for d in _idx4(src_hbm, dst_hbm, bA0 + 4 * GSTEP, bA0 + 5 * GSTEP,
                       *setA, sem_ia):
            d.start()
        for d in _idx4(src_hbm, dst_hbm, bB0, bB0 + GSTEP, *setB, sem_ib):
            d.wait()
        gB0 = [pltpu.async_copy(u_hbm.at[siB0.at[j]],
                                rb0.at[pl.ds(j * 128, 128)], sem_g0)
               for j in range(GSTEP)]
        gB1 = [pltpu.async_copy(u_hbm.at[siB1.at[j]],
                                rb1.at[pl.ds(j * 128, 128)], sem_g1)
               for j in range(GSTEP)]
        for d in gB0:
            d.wait()
        sB0 = [pltpu.async_copy(rb0.at[pl.ds(j * 128, 128)],
                                acc.at[diB0.at[j]], sem_s, add=True)
               for j in range(GSTEP)]
        for d in gB1:
            d.wait()
        sB1 = [pltpu.async_copy(rb1.at[pl.ds(j * 128, 128)],
                                acc.at[diB1.at[j]], sem_s, add=True)
               for j in range(GSTEP)]
        for d in sB0 + sB1:
            d.wait()
        return carry

    lax.fori_loop(0, NQUAD, quad, 0)
    for d in _idx4(src_hbm, dst_hbm, g0, g0 + GSTEP, *setA, sem_ia):
        d.wait()


def _dense4(p_prev, a_hbm, b_hbm, arena, r, sem):
    return [
        pltpu.make_async_copy(p_prev.at[0, pl.ds(r, DCH)],
                              arena.at[pl.ds(0, DCH)], sem),
        pltpu.make_async_copy(p_prev.at[1, pl.ds(r, DCH)],
                              arena.at[pl.ds(DCH, DCH)], sem),
        pltpu.make_async_copy(a_hbm.at[pl.ds(r, DCH)],
                              arena.at[pl.ds(2 * DCH, DCH)], sem),
        pltpu.make_async_copy(b_hbm.at[pl.ds(r, DCH)],
                              arena.at[pl.ds(3 * DCH, DCH)], sem),
    ]


def _dense_compute(arena):
    def rowfn(i, c2):
        arena[i] = arena[2 * DCH + i] * (arena[i] + arena[DCH + i]) \
            + arena[3 * DCH + i]
        return c2

    lax.fori_loop(0, DCH, rowfn, 0)


def _sc_pass_body(p_prev, a_hbm, b_hbm, src_hbm, dst_hbm, zeros_hbm,
                  p_hbm, u_hbm,
                  acc, siA0, diA0, siA1, diA1, siB0, diB0, siB1, diB1,
                  rb0, rb1,
                  sem_g0, sem_g1, sem_s, sem_ia, sem_ib,
                  sem_d0, sem_d1, sem_w0, sem_w1):
    cid = lax.axis_index("c")
    sid = lax.axis_index("s")
    row0 = sid * ROWS_PER_SUB
    _zero_acc(zeros_hbm, acc, row0)

    for d in _dense4(p_prev, a_hbm, b_hbm, rb0, row0, sem_d0):
        d.start()

    def dense_pair(t, carry):
        r0 = row0 + (2 * t) * DCH
        r1 = r0 + DCH
        rn0 = jnp.minimum(r0 + 2 * DCH, NP - DCH)
        for d in _dense4(p_prev, a_hbm, b_hbm, rb0, r0, sem_d0):
            d.wait()
        for d in _dense4(p_prev, a_hbm, b_hbm, rb1, r1, sem_d1):
            d.start()
        _dense_compute(rb0)
        w0 = pltpu.async_copy(rb0.at[pl.ds(0, DCH)],
                              u_hbm.at[pl.ds(r0, DCH)], sem_w0)
        for d in _dense4(p_prev, a_hbm, b_hbm, rb1, r1, sem_d1):
            d.wait()
        _dense_compute(rb1)
        w1 = pltpu.async_copy(rb1.at[pl.ds(0, DCH)],
                              u_hbm.at[pl.ds(r1, DCH)], sem_w1)
        w0.wait()
        for d in _dense4(p_prev, a_hbm, b_hbm, rb0, rn0, sem_d0):
            d.start()
        w1.wait()
        return carry

    lax.fori_loop(0, NDP, dense_pair, 0)
    for d in _dense4(p_prev, a_hbm, b_hbm, rb0, row0, sem_d0):
        d.wait()
    plsc.subcore_barrier()

    wid = cid * NS + sid
    _edge_phase(u_hbm, src_hbm, dst_hbm, acc,
                siA0, diA0, siA1, diA1, siB0, diB0, siB1, diB1,
                rb0, rb1, sem_g0, sem_g1, sem_s, sem_ia, sem_ib,
                wid * GPW)

    plsc.subcore_barrier()
    pltpu.sync_copy(acc.at[pl.ds(row0, ROWS_PER_SUB)],
                    p_hbm.at[cid, pl.ds(row0, ROWS_PER_SUB)])


def _sc_deg_body(dst_hbm, ones_hbm, zeros_hbm, p_hbm,
                 acc, di0, di1, rb0, sem_s):
    cid = lax.axis_index("c")
    sid = lax.axis_index("s")
    row0 = sid * ROWS_PER_SUB
    _zero_acc(zeros_hbm, acc, row0)
    pltpu.sync_copy(ones_hbm.at[pl.ds(0, GSTEP * 128)], rb0)
    plsc.subcore_barrier()

    wid = cid * NS + sid
    g0 = wid * GPW

    def pair(t, carry):
        base0 = g0 + t * 2 * GSTEP
        base1 = base0 + GSTEP
        pltpu.sync_copy(dst_hbm.at[pl.ds(base0, GSTEP)], di0)
        sd0 = [pltpu.async_copy(rb0.at[pl.ds(j * 128, 128)],
                                acc.at[di0.at[j]], sem_s, add=True)
               for j in range(GSTEP)]
        pltpu.sync_copy(dst_hbm.at[pl.ds(base1, GSTEP)], di1)
        sd1 = [pltpu.async_copy(rb0.at[pl.ds(j * 128, 128)],
                                acc.at[di1.at[j]], sem_s, add=True)
               for j in range(GSTEP)]
        for d in sd0 + sd1:
            d.wait()
        return carry

    lax.fori_loop(0, NPAIR, pair, 0)

    plsc.subcore_barrier()
    pltpu.sync_copy(acc.at[pl.ds(row0, ROWS_PER_SUB)],
                    p_hbm.at[cid, pl.ds(row0, ROWS_PER_SUB)])


def _sc_mesh():
    return plsc.VectorSubcoreMesh(core_axis_name="c", subcore_axis_name="s",
                                  num_cores=NC, num_subcores=NS)


@functools.cache
def _get_sc_pass():
    return functools.partial(
        pl.kernel,
        out_type=[
            jax.ShapeDtypeStruct((NC, NP, C), jnp.float32),
            jax.ShapeDtypeStruct((NP, C), jnp.float32),
        ],
        mesh=_sc_mesh(),
        scratch_types=(
            [pltpu.VMEM_SHARED((NP, C), jnp.float32)]
            + [pltpu.VMEM((GSTEP, 128), jnp.int32)] * 8
            + [pltpu.VMEM((GSTEP * 128, C), jnp.float32)] * 2
            + [pltpu.SemaphoreType.DMA] * 9
        ),
        compiler_params=pltpu.CompilerParams(use_tc_tiling_on_sc=False),
    )(_sc_pass_body)


@functools.cache
def _get_sc_deg():
    return functools.partial(
        pl.kernel,
        out_type=jax.ShapeDtypeStruct((NC, NP, C), jnp.float32),
        mesh=_sc_mesh(),
        scratch_types=[
            pltpu.VMEM_SHARED((NP, C), jnp.float32),
            pltpu.VMEM((GSTEP, 128), jnp.int32),
            pltpu.VMEM((GSTEP, 128), jnp.int32),
            pltpu.VMEM((GSTEP * 128, C), jnp.float32),
            pltpu.SemaphoreType.DMA,
        ],
        compiler_params=pltpu.CompilerParams(use_tc_tiling_on_sc=False),
    )(_sc_deg_body)


MLP_BLK = 4352


def _mlp_body(x_ref, wb_ref, bb_ref, w0_ref, b0_ref, wo_ref, bo_ref,
              sc_ref, h_ref):
    xb = x_ref[...]
    s = jnp.dot(xb, wb_ref[...], preferred_element_type=jnp.float32) + bb_ref[...]
    sc_ref[...] = s
    m = jnp.max(s, axis=1, keepdims=True)
    e = jnp.exp(s - m)
    sm = e / jnp.sum(e, axis=1, keepdims=True)
    h1 = jnp.dot(sm, w0_ref[...], preferred_element_type=jnp.float32) + b0_ref[...]
    h1 = jnp.maximum(h1, 0.0)
    h_ref[...] = jnp.dot(h1, wo_ref[...], preferred_element_type=jnp.float32) + bo_ref[...]


def _mlp_call(x_pad, W_base, b_base, W0, b0, W_out, b_out):
    full = lambda shape: pl.BlockSpec(shape, lambda i: (0, 0))
    return pl.pallas_call(
        _mlp_body,
        grid=(NP // MLP_BLK,),
        in_specs=[
            pl.BlockSpec((MLP_BLK, D), lambda i: (i, 0)),
            full((D, C)), full((1, C)), full((C, HID)), full((1, HID)),
            full((HID, C)), full((1, C)),
        ],
        out_specs=[
            pl.BlockSpec((MLP_BLK, C), lambda i: (i, 0)),
            pl.BlockSpec((MLP_BLK, C), lambda i: (i, 0)),
        ],
        out_shape=[
            jax.ShapeDtypeStruct((NP, C), jnp.float32),
            jax.ShapeDtypeStruct((NP, C), jnp.float32),
        ],
    )(x_pad, W_base, b_base.reshape(1, C), W0, b0.reshape(1, HID),
      W_out, b_out.reshape(1, C))


PREP_BLK = 3128


def _rows_spec(blk):
    return pl.BlockSpec((blk, 128), lambda i: (i, 0))


def _prep_body(p0_ref, p1_ref, h0_ref, b1_ref, a2_ref, b2_ref, inv_ref):
    deg = p0_ref[...] + p1_ref[...]
    dmax = jnp.where(deg > 0.0, deg, 1.0)
    u0 = h0_ref[...] * lax.rsqrt(dmax)
    b1_ref[...] = u0
    a2_ref[...] = (1.0 - ALPHA) / dmax
    b2_ref[...] = ALPHA * u0
    inv_ref[...] = jnp.sqrt(dmax)


def _prep_call(p0, p1, h0):
    sh = jax.ShapeDtypeStruct((LANE_ROWS, 128), jnp.float32)
    return pl.pallas_call(
        _prep_body,
        grid=(LANE_ROWS // PREP_BLK,),
        in_specs=[_rows_spec(PREP_BLK)] * 3,
        out_specs=[_rows_spec(PREP_BLK)] * 4,
        out_shape=[sh, sh, sh, sh],
    )(p0, p1, h0)


def _final_body(p0_ref, p1_ref, a2_ref, b2_ref, inv_ref, o_ref):
    agg = p0_ref[...] + p1_ref[...]
    o_ref[...] = (a2_ref[...] * agg + b2_ref[...]) * inv_ref[...]


def _final_call(p0, p1, a2, b2, inv):
    sh = jax.ShapeDtypeStruct((LANE_ROWS, 128), jnp.float32)
    return pl.pallas_call(
        _final_body,
        grid=(LANE_ROWS // PREP_BLK,),
        in_specs=[_rows_spec(PREP_BLK)] * 5,
        out_specs=_rows_spec(PREP_BLK),
        out_shape=sh,
    )(p0, p1, a2, b2, inv)


def kernel(x, edge_index, W_base, b_base, W0, b0, W_out, b_out):
    x_pad = jnp.pad(x, ((0, NP - N), (0, 0)))
    scores_p, h0 = _mlp_call(x_pad, W_base, b_base, W0, b0, W_out, b_out)

    npad = NGA * 128 - E
    pad_idx = jnp.full((npad,), N, jnp.int32)
    src = jnp.concatenate([edge_index[0], pad_idx]).reshape(NGA, 128)
    dst = jnp.concatenate([edge_index[1], pad_idx]).reshape(NGA, 128)
    zeros_rows = jnp.zeros((ROWS_PER_SUB, C), jnp.float32)
    ones_rows = jnp.ones((GSTEP * 128, C), jnp.float32)
    a_zero = jnp.zeros((NP, C), jnp.float32)

    r = lambda a: a.reshape(LANE_ROWS, 128)
    rr = lambda a: a.reshape(NP, C)

    pdeg = _get_sc_deg()(dst, ones_rows, zeros_rows)
    b1, a2, b2, inv = _prep_call(r(pdeg[0]), r(pdeg[1]), r(h0))

    sc_pass = _get_sc_pass()
    p = pdeg
    for k in range(K):
        a_tab = a_zero if k == 0 else rr(a2)
        b_tab = rr(b1) if k == 0 else rr(b2)
        p, _u = sc_pass(p, a_tab, b_tab, src, dst, zeros_rows)

    hK = _final_call(r(p[0]), r(p[1]), a2, b2, inv)
    adjust = hK.reshape(NP, C)[:N]
    return (adjust, scores_p[:N])


# speedup vs baseline: 1.0025x; 1.0025x over previous
import functools

import jax
import jax.numpy as jnp
from jax import lax
from jax.experimental import pallas as pl
from jax.experimental.pallas import tpu as pltpu
from jax.experimental.pallas import tpu_sc as plsc

N = 100000
E = 3200000
D = 128
C = 16
HID = 64
K = 10
ALPHA = 0.1

NC = 2
NS = 16
NW = NC * NS

NP = 100096
ROWS_PER_SUB = NP // NS
GPW = 784
EP = NW * GPW * 128
NG = EP // 128
GSTEP = 4
NQUAD = GPW // (4 * GSTEP)
NGA = NG + 8
NPAIR = GPW // (2 * GSTEP)
DCH = 92
NDP = (ROWS_PER_SUB // DCH) // 2
LANE_ROWS = NP * C // 128


def _zero_acc(zeros_hbm, acc, row0):
    pltpu.sync_copy(zeros_hbm.at[pl.ds(0, ROWS_PER_SUB)],
                    acc.at[pl.ds(row0, ROWS_PER_SUB)])


def _idx4(src_hbm, dst_hbm, b0, b1, sbuf0, dbuf0, sbuf1, dbuf1, sem):
    return [
        pltpu.make_async_copy(src_hbm.at[pl.ds(b0, GSTEP)], sbuf0, sem),
        pltpu.make_async_copy(dst_hbm.at[pl.ds(b0, GSTEP)], dbuf0, sem),
        pltpu.make_async_copy(src_hbm.at[pl.ds(b1, GSTEP)], sbuf1, sem),
        pltpu.make_async_copy(dst_hbm.at[pl.ds(b1, GSTEP)], dbuf1, sem),
    ]


def _edge_phase(u_hbm, src_hbm, dst_hbm, acc,
                siA0, diA0, siA1, diA1, siB0, diB0, siB1, diB1,
                rb0, rb1, sem_g0, sem_g1, sem_s, sem_ia, sem_ib, g0):
    setA = (siA0, diA0, siA1, diA1)
    setB = (siB0, diB0, siB1, diB1)

    def sc8_wait():
        for j in range(GSTEP):
            pltpu.make_async_copy(rb0.at[pl.ds(j * 128, 128)],
                                  acc.at[diB0.at[j]], sem_s).wait()
            pltpu.make_async_copy(rb1.at[pl.ds(j * 128, 128)],
                                  acc.at[diB1.at[j]], sem_s).wait()

    def quad(t, drain_prev):
        bA0 = g0 + t * 4 * GSTEP
        bB0 = bA0 + 2 * GSTEP
        for d in _idx4(src_hbm, dst_hbm, bA0, bA0 + GSTEP, *setA, sem_ia):
            d.wait()
        if drain_prev:
            sc8_wait()
        for d in _idx4(src_hbm, dst_hbm, bB0, bB0 + GSTEP, *setB, sem_ib):
            d.start()
        gA0 = [pltpu.async_copy(u_hbm.at[siA0.at[j]],
                                rb0.at[pl.ds(j * 128, 128)], sem_g0)
               for j in range(GSTEP)]
        gA1 = [pltpu.async_copy(u_hbm.at[siA1.at[j]],
                                rb1.at[pl.ds(j * 128, 128)], sem_g1)
               for j in range(GSTEP)]
        for d in gA0:
            d.wait()
        sA0 = [pltpu.async_copy(rb0.at[pl.ds(j * 128, 128)],
                                acc.at[diA0.at[j]], sem_s, add=True)
               for j in range(GSTEP)]
        for d in gA1:
            d.wait()
        sA1 = [pltpu.async_copy(rb1.at[pl.ds(j * 128, 128)],
                                acc.at[diA1.at[j]], sem_s, add=True)
               for j in range(GSTEP)]
        for d in _idx4(src_hbm, dst_hbm, bA0 + 4 * GSTEP, bA0 + 5 * GSTEP,
                       *setA, sem_ia):
            d.start()
        for d in sA0 + sA1:
            d.wait()
        for d in _idx4(src_hbm, dst_hbm, bB0, bB0 + GSTEP, *setB, sem_ib):
            d.wait()
        gB0 = [pltpu.async_copy(u_hbm.at[siB0.at[j]],
                                rb0.at[pl.ds(j * 128, 128)], sem_g0)
               for j in range(GSTEP)]
        gB1 = [pltpu.async_copy(u_hbm.at[siB1.at[j]],
                                rb1.at[pl.ds(j * 128, 128)], sem_g1)
               for j in range(GSTEP)]
        for d in gB0:
            d.wait()
        for j in range(GSTEP):
            pltpu.async_copy(rb0.at[pl.ds(j * 128, 128)],
                             acc.at[diB0.at[j]], sem_s, add=True)
        for d in gB1:
            d.wait()
        for j in range(GSTEP):
            pltpu.async_copy(rb1.at[pl.ds(j * 128, 128)],
                             acc.at[diB1.at[j]], sem_s, add=True)
        return 0

    for d in _idx4(src_hbm, dst_hbm, g0, g0 + GSTEP, *setA, sem_ia):
        d.start()
    quad(0, False)
    lax.fori_loop(1, NQUAD, lambda t, c: quad(t, True), 0)
    sc8_wait()
    for d in _idx4(src_hbm, dst_hbm, g0, g0 + GSTEP, *setA, sem_ia):
        d.wait()


def _dense4(p_prev, a_hbm, b_hbm, arena, r, sem):
    return [
        pltpu.make_async_copy(p_prev.at[0, pl.ds(r, DCH)],
                              arena.at[pl.ds(0, DCH)], sem),
        pltpu.make_async_copy(p_prev.at[1, pl.ds(r, DCH)],
                              arena.at[pl.ds(DCH, DCH)], sem),
        pltpu.make_async_copy(a_hbm.at[pl.ds(r, DCH)],
                              arena.at[pl.ds(2 * DCH, DCH)], sem),
        pltpu.make_async_copy(b_hbm.at[pl.ds(r, DCH)],
                              arena.at[pl.ds(3 * DCH, DCH)], sem),
    ]


def _dense_compute(arena):
    def rowfn(i, c2):
        arena[i] = arena[2 * DCH + i] * (arena[i] + arena[DCH + i]) \
            + arena[3 * DCH + i]
        return c2

    lax.fori_loop(0, DCH, rowfn, 0)


def _sc_pass_body(p_prev, a_hbm, b_hbm, src_hbm, dst_hbm, zeros_hbm,
                  p_hbm, u_hbm,
                  acc, siA0, diA0, siA1, diA1, siB0, diB0, siB1, diB1,
                  rb0, rb1,
                  sem_g0, sem_g1, sem_s, sem_ia, sem_ib,
                  sem_d0, sem_d1, sem_w0, sem_w1):
    cid = lax.axis_index("c")
    sid = lax.axis_index("s")
    row0 = sid * ROWS_PER_SUB
    _zero_acc(zeros_hbm, acc, row0)

    for d in _dense4(p_prev, a_hbm, b_hbm, rb0, row0, sem_d0):
        d.start()

    def dense_pair(t, carry):
        r0 = row0 + (2 * t) * DCH
        r1 = r0 + DCH
        rn0 = jnp.minimum(r0 + 2 * DCH, NP - DCH)
        for d in _dense4(p_prev, a_hbm, b_hbm, rb0, r0, sem_d0):
            d.wait()
        for d in _dense4(p_prev, a_hbm, b_hbm, rb1, r1, sem_d1):
            d.start()
        _dense_compute(rb0)
        w0 = pltpu.async_copy(rb0.at[pl.ds(0, DCH)],
                              u_hbm.at[pl.ds(r0, DCH)], sem_w0)
        for d in _dense4(p_prev, a_hbm, b_hbm, rb1, r1, sem_d1):
            d.wait()
        _dense_compute(rb1)
        w1 = pltpu.async_copy(rb1.at[pl.ds(0, DCH)],
                              u_hbm.at[pl.ds(r1, DCH)], sem_w1)
        w0.wait()
        for d in _dense4(p_prev, a_hbm, b_hbm, rb0, rn0, sem_d0):
            d.start()
        w1.wait()
        return carry

    lax.fori_loop(0, NDP, dense_pair, 0)
    for d in _dense4(p_prev, a_hbm, b_hbm, rb0, row0, sem_d0):
        d.wait()
    plsc.subcore_barrier()

    wid = cid * NS + sid
    _edge_phase(u_hbm, src_hbm, dst_hbm, acc,
                siA0, diA0, siA1, diA1, siB0, diB0, siB1, diB1,
                rb0, rb1, sem_g0, sem_g1, sem_s, sem_ia, sem_ib,
                wid * GPW)

    plsc.subcore_barrier()
    pltpu.sync_copy(acc.at[pl.ds(row0, ROWS_PER_SUB)],
                    p_hbm.at[cid, pl.ds(row0, ROWS_PER_SUB)])


def _sc_deg_body(dst_hbm, ones_hbm, zeros_hbm, p_hbm,
                 acc, di0, di1, rb0, sem_s, sem_i0, sem_i1):
    cid = lax.axis_index("c")
    sid = lax.axis_index("s")
    row0 = sid * ROWS_PER_SUB
    _zero_acc(zeros_hbm, acc, row0)
    pltpu.sync_copy(ones_hbm.at[pl.ds(0, GSTEP * 128)], rb0)
    plsc.subcore_barrier()

    wid = cid * NS + sid
    g0 = wid * GPW

    def ld(base, buf, sem):
        return pltpu.make_async_copy(dst_hbm.at[pl.ds(base, GSTEP)], buf, sem)

    ld(g0, di0, sem_i0).start()

    def pair(t, carry):
        base0 = g0 + t * 2 * GSTEP
        base1 = base0 + GSTEP
        ld(base0, di0, sem_i0).wait()
        ld(base1, di1, sem_i1).start()
        sd0 = [pltpu.async_copy(rb0.at[pl.ds(j * 128, 128)],
                                acc.at[di0.at[j]], sem_s, add=True)
               for j in range(GSTEP)]
        ld(base1, di1, sem_i1).wait()
        for d in sd0:
            d.wait()
        ld(base0 + 2 * GSTEP, di0, sem_i0).start()
        sd1 = [pltpu.async_copy(rb0.at[pl.ds(j * 128, 128)],
                                acc.at[di1.at[j]], sem_s, add=True)
               for j in range(GSTEP)]
        for d in sd1:
            d.wait()
        return carry

    lax.fori_loop(0, NPAIR, pair, 0)
    ld(g0, di0, sem_i0).wait()

    plsc.subcore_barrier()
    pltpu.sync_copy(acc.at[pl.ds(row0, ROWS_PER_SUB)],
                    p_hbm.at[cid, pl.ds(row0, ROWS_PER_SUB)])


def _sc_mesh():
    return plsc.VectorSubcoreMesh(core_axis_name="c", subcore_axis_name="s",
                                  num_cores=NC, num_subcores=NS)


@functools.cache
def _get_sc_pass():
    return functools.partial(
        pl.kernel,
        out_type=[
            jax.ShapeDtypeStruct((NC, NP, C), jnp.float32),
            jax.ShapeDtypeStruct((NP, C), jnp.float32),
        ],
        mesh=_sc_mesh(),
        scratch_types=(
            [pltpu.VMEM_SHARED((NP, C), jnp.float32)]
            + [pltpu.VMEM((GSTEP, 128), jnp.int32)] * 8
            + [pltpu.VMEM((GSTEP * 128, C), jnp.float32)] * 2
            + [pltpu.SemaphoreType.DMA] * 9
        ),
        compiler_params=pltpu.CompilerParams(use_tc_tiling_on_sc=False),
    )(_sc_pass_body)


@functools.cache
def _get_sc_deg():
    return functools.partial(
        pl.kernel,
        out_type=jax.ShapeDtypeStruct((NC, NP, C), jnp.float32),
        mesh=_sc_mesh(),
        scratch_types=[
            pltpu.VMEM_SHARED((NP, C), jnp.float32),
            pltpu.VMEM((GSTEP, 128), jnp.int32),
            pltpu.VMEM((GSTEP, 128), jnp.int32),
            pltpu.VMEM((GSTEP * 128, C), jnp.float32),
            pltpu.SemaphoreType.DMA,
            pltpu.SemaphoreType.DMA,
            pltpu.SemaphoreType.DMA,
        ],
        compiler_params=pltpu.CompilerParams(use_tc_tiling_on_sc=False),
    )(_sc_deg_body)


MLP_BLK = 4000


def _mlp_body(x_ref, wb_ref, bb_ref, w0_ref, b0_ref, wo_ref, bo_ref,
              sc_ref, h_ref):
    xb = x_ref[...]
    s = jnp.dot(xb, wb_ref[...], preferred_element_type=jnp.float32) + bb_ref[...]
    sc_ref[...] = s
    m = jnp.max(s, axis=1, keepdims=True)
    e = jnp.exp(s - m)
    sm = e / jnp.sum(e, axis=1, keepdims=True)
    h1 = jnp.dot(sm, w0_ref[...], preferred_element_type=jnp.float32) + b0_ref[...]
    h1 = jnp.maximum(h1, 0.0)
    h_ref[...] = jnp.dot(h1, wo_ref[...], preferred_element_type=jnp.float32) + bo_ref[...]


def _mlp_call(x_pad, W_base, b_base, W0, b0, W_out, b_out):
    full = lambda shape: pl.BlockSpec(shape, lambda i: (0, 0))
    return pl.pallas_call(
        _mlp_body,
        grid=(N // MLP_BLK,),
        in_specs=[
            pl.BlockSpec((MLP_BLK, D), lambda i: (i, 0)),
            full((D, C)), full((1, C)), full((C, HID)), full((1, HID)),
            full((HID, C)), full((1, C)),
        ],
        out_specs=[
            pl.BlockSpec((MLP_BLK, C), lambda i: (i, 0)),
            pl.BlockSpec((MLP_BLK, C), lambda i: (i, 0)),
        ],
        out_shape=[
            jax.ShapeDtypeStruct((N, C), jnp.float32),
            jax.ShapeDtypeStruct((N, C), jnp.float32),
        ],
    )(x_pad, W_base, b_base.reshape(1, C), W0, b0.reshape(1, HID),
      W_out, b_out.reshape(1, C))


PREP_BLK = 3128


def _rows_spec(blk):
    return pl.BlockSpec((blk, 128), lambda i: (i, 0))


def _prep_body(p0_ref, p1_ref, h0_ref, b1_ref, a2_ref, b2_ref, inv_ref):
    deg = p0_ref[...] + p1_ref[...]
    dmax = jnp.where(deg > 0.0, deg, 1.0)
    u0 = h0_ref[...] * lax.rsqrt(dmax)
    b1_ref[...] = u0
    a2_ref[...] = (1.0 - ALPHA) / dmax
    b2_ref[...] = ALPHA * u0
    inv_ref[...] = jnp.sqrt(dmax)


def _prep_call(p0, p1, h0):
    sh = jax.ShapeDtypeStruct((LANE_ROWS, 128), jnp.float32)
    return pl.pallas_call(
        _prep_body,
        grid=(LANE_ROWS // PREP_BLK,),
        in_specs=[_rows_spec(PREP_BLK)] * 3,
        out_specs=[_rows_spec(PREP_BLK)] * 4,
        out_shape=[sh, sh, sh, sh],
    )(p0, p1, h0)


def _final_body(p0_ref, p1_ref, a2_ref, b2_ref, inv_ref, o_ref):
    agg = p0_ref[...] + p1_ref[...]
    o_ref[...] = (a2_ref[...] * agg + b2_ref[...]) * inv_ref[...]


def _final_call(p0, p1, a2, b2, inv):
    sh = jax.ShapeDtypeStruct((LANE_ROWS, 128), jnp.float32)
    return pl.pallas_call(
        _final_body,
        grid=(LANE_ROWS // PREP_BLK,),
        in_specs=[_rows_spec(PREP_BLK)] * 5,
        out_specs=_rows_spec(PREP_BLK),
        out_shape=sh,
    )(p0, p1, a2, b2, inv)


def kernel(x, edge_index, W_base, b_base, W0, b0, W_out, b_out):
    scores, h0 = _mlp_call(x, W_base, b_base, W0, b0, W_out, b_out)
    h0p = jnp.pad(h0, ((0, NP - N), (0, 0)))

    npad = NGA * 128 - E
    pad_idx = jnp.full((npad,), N, jnp.int32)
    src = jnp.concatenate([edge_index[0], pad_idx]).reshape(NGA, 128)
    dst = jnp.concatenate([edge_index[1], pad_idx]).reshape(NGA, 128)
    zeros_rows = jnp.zeros((ROWS_PER_SUB, C), jnp.float32)
    ones_rows = jnp.ones((GSTEP * 128, C), jnp.float32)
    a_zero = jnp.zeros((NP, C), jnp.float32)

    r = lambda a: a.reshape(LANE_ROWS, 128)
    rr = lambda a: a.reshape(NP, C)

    pdeg = _get_sc_deg()(dst, ones_rows, zeros_rows)
    b1, a2, b2, inv = _prep_call(r(pdeg[0]), r(pdeg[1]), r(h0p))

    sc_pass = _get_sc_pass()
    p = pdeg
    for k in range(K):
        a_tab = a_zero if k == 0 else rr(a2)
        b_tab = rr(b1) if k == 0 else rr(b2)
        p, _u = sc_pass(p, a_tab, b_tab, src, dst, zeros_rows)

    hK = _final_call(r(p[0]), r(p[1]), a2, b2, inv)
    adjust = hK.reshape(NP, C)[:N]
    return (adjust, scores)
